# baseline (speedup 1.0000x reference)
import functools

import jax
import jax.numpy as jnp
from jax import lax
from jax.experimental import pallas as pl
from jax.experimental.pallas import tpu as pltpu
from jax.experimental.pallas import tpu_sc as plsc

_D = 256
_V = 16
_NC, _NS = 2, 16
_NW = _NC * _NS
_CH = 64
_NBUF = 2
_L = 16


def _emb_body(bpw, ids_hbm, table_hbm, out_hbm, idx_v, table_v, rows_v, ssem):
    nchunk = bpw // _CH
    wid = lax.axis_index("s") * _NC + lax.axis_index("c")
    base = wid * bpw

    pltpu.sync_copy(table_hbm, table_v)
    pltpu.sync_copy(ids_hbm.at[pl.ds(base, bpw)], idx_v)

    _CHD = _CH * _D

    def build(t, bstatic):
        pending = None
        for g in range(_CH // _L):
            ids16 = idx_v[pl.ds(t * _CH + g * _L, _L)] * _D
            rbs = [ids16[r] for r in range(_L)]
            for r in range(_L):
                row_off = bstatic * _CHD + (g * _L + r) * _D
                vecs = [table_v[pl.ds(rbs[r] + j * _L, _L)]
                        for j in range(_D // _L)]
                if pending is not None:
                    poff, pvecs = pending
                    for j in range(_D // _L):
                        rows_v[pl.ds(poff + j * _L, _L)] = pvecs[j]
                pending = (row_off, vecs)
        poff, pvecs = pending
        for j in range(_D // _L):
            rows_v[pl.ds(poff + j * _L, _L)] = pvecs[j]

    def drain(b):
        pltpu.make_async_copy(
            rows_v.at[pl.ds(b * _CHD, _CHD)],
            out_hbm.at[pl.ds(base * _D, _CHD)], ssem.at[b]).wait()

    def step(t, carry):
        b = lax.rem(t, _NBUF)

        @pl.when(t >= _NBUF)
        def _():
            drain(b)

        for bs in range(_NBUF):
            @pl.when(b == bs)
            def _():
                build(t, bs)

        pltpu.async_copy(
            rows_v.at[pl.ds(b * _CHD, _CHD)],
            out_hbm.at[pl.ds((base + t * _CH) * _D, _CHD)], ssem.at[b])
        return carry

    lax.fori_loop(0, nchunk, step, 0)
    for b in range(_NBUF):
        drain(b)


@functools.partial(jax.jit, static_argnums=(2,))
def _emb(flat_ids, flat_table, n):
    bpw = n // _NW
    grid_kernel = functools.partial(
        pl.kernel,
        out_type=jax.ShapeDtypeStruct((n * _D,), jnp.float32),
        mesh=plsc.VectorSubcoreMesh(core_axis_name="c", subcore_axis_name="s"),
        compiler_params=pltpu.CompilerParams(needs_layout_passes=False),
        scratch_types=[
            pltpu.VMEM((bpw,), jnp.int32),
            pltpu.VMEM((_V * _D,), jnp.float32),
            pltpu.VMEM((_NBUF * _CH * _D,), jnp.float32),
            pltpu.SemaphoreType.DMA((_NBUF,)),
        ],
    )
    return grid_kernel(functools.partial(_emb_body, bpw))(flat_ids, flat_table)


def kernel(input_ids, table):
    n = input_ids.size
    flat = input_ids.reshape((n,))
    out = _emb(flat, table.reshape((-1,)), n)
    return out.reshape(input_ids.shape + (table.shape[1],))
